# initial kernel scaffold (unmeasured)
import jax
import jax.numpy as jnp
from jax import lax
from jax.experimental import pallas as pl
from jax.experimental.pallas import tpu as pltpu

N_DEV = 32
M_PER = 128
K = 4096
N = 2048
K_PER = 128


def kernel(x, w_mat):
    def body(x_ref, w_ref, out_ref, stage_ref, xg_ref, send_sems, recv_sems):
        my = lax.axis_index("i")

        for d in range(N_DEV):
            dst = (my + d) % N_DEV
            stage_ref[d, :, :] = x_ref[pl.ds(dst * M_PER, M_PER), :].astype(
                jnp.bfloat16
            )

        barrier_sem = pltpu.get_barrier_semaphore()
        for d in range(1, N_DEV):
            dst = (my + d) % N_DEV
            pl.semaphore_signal(
                barrier_sem, inc=1,
                device_id=(dst,), device_id_type=pl.DeviceIdType.MESH,
            )
        pl.semaphore_wait(barrier_sem, N_DEV - 1)

        sends = []
        for d in range(1, N_DEV):
            dst = (my + d) % N_DEV
            rdma = pltpu.make_async_remote_copy(
                src_ref=stage_ref.at[d],
                dst_ref=xg_ref.at[d],
                send_sem=send_sems.at[d],
                recv_sem=recv_sems.at[d],
                device_id=(dst,),
                device_id_type=pl.DeviceIdType.MESH,
            )
            rdma.start()
            sends.append(rdma)

        xg_ref[0, :, :] = stage_ref[0, :, :]

        dims = (((1,), (0,)), ((), ()))
        for d in range(N_DEV):
            if d > 0:
                recv = pltpu.make_async_remote_copy(
                    src_ref=stage_ref.at[d],
                    dst_ref=xg_ref.at[d],
                    send_sem=send_sems.at[d],
                    recv_sem=recv_sems.at[d],
                    device_id=(my,),
                    device_id_type=pl.DeviceIdType.MESH,
                )
                recv.wait_recv()
            j = (my - d) % N_DEV
            a = xg_ref[d, :, :]
            b = w_ref[pl.ds(j * K_PER, K_PER), :].astype(jnp.bfloat16)
            partial = lax.dot_general(
                a, b, dims, preferred_element_type=jnp.float32
            )
            if d == 0:
                out_ref[:, :] = partial
            else:
                out_ref[:, :] += partial

        for rdma in sends:
            rdma.wait_send()

    return pl.pallas_call(
        body,
        out_shape=jax.ShapeDtypeStruct((M_PER, N), jnp.float32),
        in_specs=[
            pl.BlockSpec(memory_space=pltpu.VMEM),
            pl.BlockSpec(memory_space=pltpu.VMEM),
        ],
        out_specs=pl.BlockSpec(memory_space=pltpu.VMEM),
        scratch_shapes=[
            pltpu.VMEM((N_DEV, M_PER, K_PER), jnp.bfloat16),
            pltpu.VMEM((N_DEV, M_PER, K_PER), jnp.bfloat16),
            pltpu.SemaphoreType.DMA((N_DEV,)),
            pltpu.SemaphoreType.DMA((N_DEV,)),
        ],
        compiler_params=pltpu.CompilerParams(collective_id=0),
    )(x, w_mat)


# baseline (device time: 39013 ns/iter reference)
import jax
import jax.numpy as jnp
from jax import lax
from jax.experimental import pallas as pl
from jax.experimental.pallas import tpu as pltpu

N_DEV = 32
M_PER = 128
K = 4096
N = 2048
K_PER = 128


def kernel(x, w_mat):
    def body(x_ref, w_ref, out_ref, stage_ref, xg_ref, send_sems, recv_sems):
        my = lax.axis_index("i")

        for d in range(N_DEV):
            dst = (my + d) % N_DEV
            stage_ref[d, :, :] = x_ref[pl.ds(dst * M_PER, M_PER), :].astype(
                jnp.bfloat16
            )

        barrier_sem = pltpu.get_barrier_semaphore()
        for d in range(1, N_DEV):
            dst = (my + d) % N_DEV
            pl.semaphore_signal(
                barrier_sem, inc=1,
                device_id=(dst,), device_id_type=pl.DeviceIdType.MESH,
            )
        pl.semaphore_wait(barrier_sem, N_DEV - 1)

        sends = []
        for d in range(1, N_DEV):
            dst = (my + d) % N_DEV
            rdma = pltpu.make_async_remote_copy(
                src_ref=stage_ref.at[d],
                dst_ref=xg_ref.at[d],
                send_sem=send_sems.at[d],
                recv_sem=recv_sems.at[d],
                device_id=(dst,),
                device_id_type=pl.DeviceIdType.MESH,
            )
            rdma.start()
            sends.append(rdma)

        xg_ref[0, :, :] = stage_ref[0, :, :]

        dims = (((1,), (0,)), ((), ()))
        for d in range(N_DEV):
            if d > 0:
                recv = pltpu.make_async_remote_copy(
                    src_ref=stage_ref.at[d],
                    dst_ref=xg_ref.at[d],
                    send_sem=send_sems.at[d],
                    recv_sem=recv_sems.at[d],
                    device_id=(my,),
                    device_id_type=pl.DeviceIdType.MESH,
                )
                recv.wait_recv()
            j = (my - d) % N_DEV
            a = xg_ref[d, :, :]
            b = w_ref[pl.ds(j * K_PER, K_PER), :].astype(jnp.bfloat16)
            partial = lax.dot_general(
                a, b, dims, preferred_element_type=jnp.float32
            )
            if d == 0:
                out_ref[:, :] = partial
            else:
                out_ref[:, :] += partial

        for rdma in sends:
            rdma.wait_send()

    return pl.pallas_call(
        body,
        out_shape=jax.ShapeDtypeStruct((M_PER, N), jnp.float32),
        in_specs=[
            pl.BlockSpec(memory_space=pltpu.VMEM),
            pl.BlockSpec(memory_space=pltpu.VMEM),
        ],
        out_specs=pl.BlockSpec(memory_space=pltpu.VMEM),
        scratch_shapes=[
            pltpu.VMEM((N_DEV, M_PER, K_PER), jnp.bfloat16),
            pltpu.VMEM((N_DEV, M_PER, K_PER), jnp.bfloat16),
            pltpu.SemaphoreType.DMA((N_DEV,)),
            pltpu.SemaphoreType.DMA((N_DEV,)),
        ],
        compiler_params=pltpu.CompilerParams(
            collective_id=0, vmem_limit_bytes=100 * 1024 * 1024
        ),
    )(x, w_mat)
